# Initial kernel scaffold; baseline (speedup 1.0000x reference)
import jax
import jax.numpy as jnp
from jax.experimental import pallas as pl


def kernel(x, edge_index, Wm1, bm1, Wm2, bm2, Wn, bn, Dw0, Db0, Dw1, Db1, Dw2, Db2):
    raise NotImplementedError("write your pallas kernel here")


# speedup vs baseline: 1.2697x; 1.2697x over previous
import jax
import jax.numpy as jnp
from jax.experimental import pallas as pl

_B = 1024
_DIM_IN = 512
_N_ATOMS = 64
_F = 4
_DIM_REST = _DIM_IN - _N_ATOMS * 3
_E = 1024
_MSG_H = 16

_BLK = 128


def _fused(xr_ref, pos_ref, ei_ref, wm1_ref, bm1_ref, wm2_ref, bm2_ref,
           wn_ref, bn_ref, dw0t_ref, dw0g_ref, db0_ref, dw1_ref, db1_ref,
           dw2_ref, db2_ref, out_ref):
    f32 = jnp.float32
    x_rest = xr_ref[...]
    pos = pos_ref[...]

    src = ei_ref[0, 0, :]
    dst = ei_ref[0, 1, :]
    atoms = jax.lax.broadcasted_iota(jnp.int32, (_E, _N_ATOMS), 1)
    oh_s = (atoms == src[:, None]).astype(f32)
    oh_d = (atoms == dst[:, None]).astype(f32)

    dn_g = (((1,), (1,)), ((), ()))
    ps = jax.lax.dot_general(pos, oh_s, dn_g, preferred_element_type=f32)
    pd = jax.lax.dot_general(pos, oh_d, dn_g, preferred_element_type=f32)
    diff = ps - pd
    d = jnp.sqrt(jnp.sum(diff * diff, axis=1) + 1e-8)

    wm1 = wm1_ref[...]
    dn_a = (((0,), (1,)), ((), ()))
    h_ps = jax.lax.dot_general(wm1[0:3, :].T, ps, (((1,), (1,)), ((), ())),
                               preferred_element_type=f32)
    h_pd = jax.lax.dot_general(wm1[3:6, :].T, pd, (((1,), (1,)), ((), ())),
                               preferred_element_type=f32)
    pre = (h_ps + h_pd + wm1[6, :][:, None, None] * d[None, :, :]
           + bm1_ref[0, :][:, None, None])
    hid = jnp.maximum(pre, 0.0)

    m = jax.lax.dot_general(wm2_ref[...], hid, (((0,), (0,)), ((), ())),
                            preferred_element_type=f32)
    m = m + bm2_ref[0, :][:, None, None]

    agg = jax.lax.dot_general(m, oh_d, (((2,), (0,)), ((), ())),
                              preferred_element_type=f32)

    wn = wn_ref[...]
    n1 = jax.lax.dot_general(wn[0:3, :], pos, (((0,), (2,)), ((), ())),
                             preferred_element_type=f32)
    n2 = jax.lax.dot_general(wn[3:7, :], agg, (((0,), (0,)), ((), ())),
                             preferred_element_type=f32)
    h = jnp.tanh(n1 + n2 + bn_ref[0, :][:, None, None])

    l0g = jax.lax.dot_general(h, dw0g_ref[...],
                              (((2,), (1,)), ((0,), (0,))),
                              preferred_element_type=f32)
    l0 = jnp.dot(x_rest, dw0t_ref[...], preferred_element_type=f32) \
        + l0g[0] + l0g[1] + l0g[2] + l0g[3] + db0_ref[0, :][None, :]
    h1 = jnp.maximum(l0, 0.0)
    h2 = jnp.maximum(jnp.dot(h1, dw1_ref[...], preferred_element_type=f32)
                     + db1_ref[0, :][None, :], 0.0)
    out_ref[...] = jnp.dot(h2, dw2_ref[...], preferred_element_type=f32) \
        + db2_ref[0, :][None, :]


def kernel(x, edge_index, Wm1, bm1, Wm2, bm2, Wn, bn, Dw0, Db0, Dw1, Db1, Dw2, Db2):
    f32 = jnp.float32
    x_rest = x[:, :_DIM_REST]
    pos = x[:, _DIM_REST:].reshape(_B, _N_ATOMS, 3)
    ei3 = edge_index.reshape(1, 2, _E)
    wm1p = jnp.concatenate([Wm1, jnp.zeros((1, _MSG_H), f32)], axis=0)
    wnp = jnp.concatenate([Wn, jnp.zeros((1, _F), f32)], axis=0)
    dw0_top = Dw0[:_DIM_REST]
    dw0_gnn = Dw0[_DIM_REST:].reshape(_N_ATOMS, _F, -1).transpose(1, 0, 2)

    grid = (_B // _BLK,)
    full = lambda shape: pl.BlockSpec(shape, lambda i: (0,) * len(shape))
    out = pl.pallas_call(
        _fused,
        grid=grid,
        in_specs=[
            pl.BlockSpec((_BLK, _DIM_REST), lambda i: (i, 0)),
            pl.BlockSpec((_BLK, _N_ATOMS, 3), lambda i: (i, 0, 0)),
            full((1, 2, _E)),
            full((8, _MSG_H)),
            full((1, _MSG_H)),
            full((_MSG_H, _F)),
            full((1, _F)),
            full((8, _F)),
            full((1, _F)),
            full(dw0_top.shape),
            full(dw0_gnn.shape),
            full((1, Db0.shape[0])),
            full(Dw1.shape),
            full((1, Db1.shape[0])),
            full(Dw2.shape),
            full((1, Db2.shape[0])),
        ],
        out_specs=pl.BlockSpec((_BLK, Dw2.shape[1]), lambda i: (i, 0)),
        out_shape=jax.ShapeDtypeStruct((_B, Dw2.shape[1]), f32),
    )(x_rest, pos, ei3, wm1p, bm1.reshape(1, -1), Wm2, bm2.reshape(1, -1),
      wnp, bn.reshape(1, -1), dw0_top, dw0_gnn, Db0.reshape(1, -1), Dw1,
      Db1.reshape(1, -1), Dw2, Db2.reshape(1, -1))
    return out
